# baseline (speedup 1.0000x reference)
import functools

import jax
import jax.numpy as jnp
from jax import lax
from jax.experimental import pallas as pl
from jax.experimental.pallas import tpu as pltpu
from jax.experimental.pallas import tpu_sc as plsc

NS = 16
NC = 2
CHUNK = 80

_HIGH = jax.lax.Precision.HIGHEST


def _sc_mesh():
    return plsc.VectorSubcoreMesh(core_axis_name="c", subcore_axis_name="s",
                                  num_cores=NC, num_subcores=NS)


def _make_deg_kernel(n_nodes, n_chunks, nacc, rows_per_tile, half):
    w_chunks = n_chunks // 2

    @functools.partial(
        pl.kernel,
        out_type=jax.ShapeDtypeStruct((2 * n_nodes, half), jnp.float32),
        mesh=_sc_mesh(),
        scratch_types=[
            pltpu.VMEM((w_chunks, CHUNK), jnp.int32),
            pltpu.VMEM((CHUNK, half), jnp.float32),
            [pltpu.SemaphoreType.DMA for _ in range(3)],
            pltpu.VMEM_SHARED((nacc, half), jnp.float32),
        ],
    )
    def deg_kernel(ones_hbm, dst_hbm, out_hbm, dst_v, ones_v, sems, acc):
        c = lax.axis_index("c")
        s = lax.axis_index("s")
        pltpu.sync_copy(dst_hbm.at[s, c], dst_v)
        pltpu.sync_copy(ones_hbm.at[pl.ds(0, CHUNK)], ones_v)
        row0 = s * rows_per_tile
        pltpu.sync_copy(ones_hbm, acc.at[pl.ds(row0, rows_per_tile)])
        plsc.subcore_barrier()

        def body(i, _):
            b = 3 * i
            for t in range(3):
                pltpu.async_copy(ones_v, acc.at[dst_v.at[b + t]], sems[t],
                                 add=True)
            for t in range(3):
                pltpu.make_async_copy(ones_v, acc.at[dst_v.at[b + t]],
                                      sems[t]).wait()
            return 0

        lax.fori_loop(0, w_chunks // 3, body, 0)
        plsc.subcore_barrier()
        pltpu.sync_copy(
            acc.at[pl.ds(row0, rows_per_tile)],
            out_hbm.at[pl.ds(c * n_nodes + row0, rows_per_tile)],
        )

    return deg_kernel


def _make_agg_kernel(n_nodes, n_chunks, nacc, rows_per_tile, half):
    W = n_chunks // 2

    @functools.partial(
        pl.kernel,
        out_type=jax.ShapeDtypeStruct((2 * n_nodes, half), jnp.float32),
        mesh=_sc_mesh(),
        scratch_types=[
            pltpu.VMEM((W, CHUNK), jnp.int32),
            pltpu.VMEM((W, CHUNK), jnp.int32),
            [pltpu.VMEM((CHUNK, half), jnp.float32) for _ in range(3)],
            [pltpu.SemaphoreType.DMA for _ in range(3)],
            [pltpu.SemaphoreType.DMA for _ in range(3)],
            pltpu.VMEM_SHARED((nacc, half), jnp.float32),
        ],
    )
    def agg_kernel(y_hbm, src_hbm, dst_hbm, out_hbm,
                   src_v, dst_v, bufs, gsem, ssem, acc):
        c = lax.axis_index("c")
        s = lax.axis_index("s")
        row0 = c * n_nodes + s * rows_per_tile
        pltpu.sync_copy(
            y_hbm.at[pl.ds(row0, rows_per_tile)],
            acc.at[pl.ds(s * rows_per_tile, rows_per_tile)],
        )
        plsc.subcore_barrier()

        def g_fire(j, t):
            pltpu.async_copy(y_hbm.at[src_v.at[j]], bufs[t], gsem[t])

        def g_wait(j, t):
            pltpu.make_async_copy(y_hbm.at[src_v.at[j]], bufs[t],
                                  gsem[t]).wait()

        def sc_fire(j, t):
            pltpu.async_copy(bufs[t], acc.at[dst_v.at[j]], ssem[t], add=True)

        def sc_wait(j, t):
            pltpu.make_async_copy(bufs[t], acc.at[dst_v.at[j]],
                                  ssem[t]).wait()

        def body(k, _):
            j = 3 * k
            g_wait(j, 0); sc_fire(j, 0)
            sc_wait(j - 1, 2); g_fire(j + 2, 2)
            g_wait(j + 1, 1); sc_fire(j + 1, 1)
            sc_wait(j, 0); g_fire(j + 3, 0)
            g_wait(j + 2, 2); sc_fire(j + 2, 2)
            sc_wait(j + 1, 1); g_fire(j + 4, 1)
            return 0

        for w in range(2):
            pltpu.sync_copy(src_hbm.at[c, s, w], src_v)
            pltpu.sync_copy(dst_hbm.at[s, w], dst_v)
            g_fire(0, 0); g_fire(1, 1); g_fire(2, 2)
            g_wait(0, 0); sc_fire(0, 0)
            g_wait(1, 1); sc_fire(1, 1)
            sc_wait(0, 0); g_fire(3, 0)
            g_wait(2, 2); sc_fire(2, 2)
            sc_wait(1, 1); g_fire(4, 1)
            lax.fori_loop(1, W // 3 - 1, body, 0)
            e = W - 3
            g_wait(e, 0); sc_fire(e, 0)
            sc_wait(e - 1, 2); g_fire(e + 2, 2)
            g_wait(e + 1, 1); sc_fire(e + 1, 1)
            sc_wait(e, 0)
            g_wait(e + 2, 2); sc_fire(e + 2, 2)
            sc_wait(e + 1, 1); sc_wait(e + 2, 2)
        plsc.subcore_barrier()
        pltpu.sync_copy(
            acc.at[pl.ds(s * rows_per_tile, rows_per_tile)],
            out_hbm.at[pl.ds(row0, rows_per_tile)],
        )

    return agg_kernel


def _scale_kernel(n_nodes, n_pad, in_ch, half, acc_w, blk):
    def body(deg_ref, x_ref, dinv_ref, y_ref):
        degp1 = deg_ref[0][:, 0:1] + deg_ref[1][:, 0:1] - 1.0
        dinv = lax.rsqrt(degp1)
        dinv_ref[...] = jnp.broadcast_to(dinv, (blk, acc_w))
        y = x_ref[...] * dinv
        y_ref[0] = y[:, :half]
        y_ref[1] = y[:, half:]

    grid = n_nodes // blk
    return pl.pallas_call(
        body,
        grid=(grid,),
        in_specs=[
            pl.BlockSpec((2, blk, half), lambda i: (0, i, 0)),
            pl.BlockSpec((blk, in_ch), lambda i: (i, 0)),
        ],
        out_specs=[
            pl.BlockSpec((blk, acc_w), lambda i: (i, 0)),
            pl.BlockSpec((2, blk, half), lambda i: (0, i, 0)),
        ],
        out_shape=[
            jax.ShapeDtypeStruct((n_pad, acc_w), jnp.float32),
            jax.ShapeDtypeStruct((2, n_pad, half), jnp.float32),
        ],
    )


def _mlp_kernel(n_nodes, n_pad, in_ch, hid_ch, out_ch, half, acc_w, blk):
    def body(agg_ref, dinv_ref, w1_ref, b1_ref, w2_ref, z_ref):
        a = jnp.concatenate([agg_ref[0], agg_ref[1]], axis=1)
        dinv = dinv_ref[...][:, 0:1]
        a = a * dinv
        h = jnp.dot(a, w1_ref[...], preferred_element_type=jnp.float32,
                    precision=None)
        h = jnp.maximum(h + b1_ref[...], 0.0)
        z = jnp.dot(h, w2_ref[...], preferred_element_type=jnp.float32,
                    precision=None)
        z = z * dinv
        z_ref[0] = z[:, :half]
        z_ref[1] = z[:, half:]

    grid = n_nodes // blk
    return pl.pallas_call(
        body,
        grid=(grid,),
        in_specs=[
            pl.BlockSpec((2, blk, half), lambda i: (0, i, 0)),
            pl.BlockSpec((blk, acc_w), lambda i: (i, 0)),
            pl.BlockSpec((in_ch, hid_ch), lambda i: (0, 0)),
            pl.BlockSpec((1, hid_ch), lambda i: (0, 0)),
            pl.BlockSpec((hid_ch, out_ch), lambda i: (0, 0)),
        ],
        out_specs=pl.BlockSpec((2, blk, half), lambda i: (0, i, 0)),
        out_shape=jax.ShapeDtypeStruct((2, n_pad, half), jnp.float32),
    )


def _final_kernel(n_nodes, out_ch, half, acc_w, blk):
    def body(agg_ref, dinv_ref, b2_ref, out_ref):
        a = jnp.concatenate([agg_ref[0], agg_ref[1]], axis=1)
        dinv = dinv_ref[...][:, 0:1]
        out_ref[...] = a * dinv + b2_ref[...]

    grid = n_nodes // blk
    return pl.pallas_call(
        body,
        grid=(grid,),
        in_specs=[
            pl.BlockSpec((2, blk, half), lambda i: (0, i, 0)),
            pl.BlockSpec((blk, acc_w), lambda i: (i, 0)),
            pl.BlockSpec((1, out_ch), lambda i: (0, 0)),
        ],
        out_specs=pl.BlockSpec((blk, out_ch), lambda i: (i, 0)),
        out_shape=jax.ShapeDtypeStruct((n_nodes, out_ch), jnp.float32),
    )


def kernel(x, edge_index, W1, b1, W2, b2):
    n_nodes, in_ch = x.shape
    hid_ch = W1.shape[1]
    out_ch = W2.shape[1]
    n_edges = edge_index.shape[1]
    half = in_ch // 2
    acc_w = 8
    blk = 2000

    n_pad = -(-n_nodes // (NS * 8)) * (NS * 8)
    rows_per_tile = n_pad // NS
    per_tile = -(-n_edges // NS)
    n_chunks = -(-per_tile // CHUNK)
    n_chunks = (n_chunks + 5) // 6 * 6
    nacc = n_pad

    src = edge_index[0].astype(jnp.int32)
    dst = edge_index[1].astype(jnp.int32)
    e_pad = NS * n_chunks * CHUNK - n_edges
    src_p = jnp.concatenate([src, jnp.zeros((e_pad,), jnp.int32)])
    dst_p = jnp.concatenate([dst, jnp.full((e_pad,), n_nodes, jnp.int32)])
    W = n_chunks // 2
    src_t = src_p.reshape(NS, 2, W, CHUNK)
    src_both = jnp.stack([src_t, src_t + n_pad])
    dst_t = dst_p.reshape(NS, 2, W, CHUNK)

    agg = _make_agg_kernel(n_pad, n_chunks, nacc, rows_per_tile, half)
    ones = jnp.ones((rows_per_tile, half), jnp.float32)
    degp = _make_deg_kernel(n_pad, n_chunks, nacc, rows_per_tile, half)(
        ones, dst_t)
    dinv, ybuf = _scale_kernel(n_nodes, n_pad, in_ch, half, acc_w, blk)(
        degp.reshape(2, n_pad, half), x)

    agg1 = agg(ybuf.reshape(2 * n_pad, half), src_both, dst_t)
    z = _mlp_kernel(n_nodes, n_pad, in_ch, hid_ch, out_ch, half, acc_w, blk)(
        agg1.reshape(2, n_pad, half), dinv, W1, b1.reshape(1, hid_ch), W2)
    agg2 = agg(z.reshape(2 * n_pad, half), src_both, dst_t)
    return _final_kernel(n_nodes, out_ch, half, acc_w, blk)(
        agg2.reshape(2, n_pad, half), dinv, b2.reshape(1, out_ch))


# speedup vs baseline: 1.0135x; 1.0135x over previous
import functools

import jax
import jax.numpy as jnp
from jax import lax
from jax.experimental import pallas as pl
from jax.experimental.pallas import tpu as pltpu
from jax.experimental.pallas import tpu_sc as plsc

NS = 16
NC = 2
CHUNK = 80

_HIGH = jax.lax.Precision.HIGHEST


def _sc_mesh():
    return plsc.VectorSubcoreMesh(core_axis_name="c", subcore_axis_name="s",
                                  num_cores=NC, num_subcores=NS)


def _make_deg_kernel(n_nodes, n_chunks, nacc, rows_per_tile, half):
    w_chunks = n_chunks // 2

    @functools.partial(
        pl.kernel,
        out_type=jax.ShapeDtypeStruct((2 * n_nodes, half), jnp.float32),
        mesh=_sc_mesh(),
        scratch_types=[
            pltpu.VMEM((w_chunks, CHUNK), jnp.int32),
            pltpu.VMEM((CHUNK, half), jnp.float32),
            [pltpu.SemaphoreType.DMA for _ in range(3)],
            pltpu.VMEM_SHARED((nacc, half), jnp.float32),
        ],
    )
    def deg_kernel(ones_hbm, dst_hbm, out_hbm, dst_v, ones_v, sems, acc):
        c = lax.axis_index("c")
        s = lax.axis_index("s")
        pltpu.sync_copy(dst_hbm.at[s, c], dst_v)
        pltpu.sync_copy(ones_hbm.at[pl.ds(0, CHUNK)], ones_v)
        row0 = s * rows_per_tile
        pltpu.sync_copy(ones_hbm, acc.at[pl.ds(row0, rows_per_tile)])
        plsc.subcore_barrier()

        def body(i, _):
            b = 3 * i
            for t in range(3):
                pltpu.async_copy(ones_v, acc.at[dst_v.at[b + t]], sems[t],
                                 add=True)
            for t in range(3):
                pltpu.make_async_copy(ones_v, acc.at[dst_v.at[b + t]],
                                      sems[t]).wait()
            return 0

        lax.fori_loop(0, w_chunks // 3, body, 0)
        plsc.subcore_barrier()
        pltpu.sync_copy(
            acc.at[pl.ds(row0, rows_per_tile)],
            out_hbm.at[pl.ds(c * n_nodes + row0, rows_per_tile)],
        )

    return deg_kernel


def _make_agg_kernel(n_nodes, n_chunks, nacc, rows_per_tile, half):
    W = n_chunks // 2

    @functools.partial(
        pl.kernel,
        out_type=jax.ShapeDtypeStruct((2 * n_nodes, half), jnp.float32),
        mesh=_sc_mesh(),
        scratch_types=[
            pltpu.VMEM((W, CHUNK), jnp.int32),
            pltpu.VMEM((W, CHUNK), jnp.int32),
            [pltpu.VMEM((CHUNK, half), jnp.float32) for _ in range(3)],
            [pltpu.SemaphoreType.DMA for _ in range(3)],
            [pltpu.SemaphoreType.DMA for _ in range(3)],
            pltpu.VMEM_SHARED((nacc, half), jnp.float32),
        ],
    )
    def agg_kernel(y_hbm, src_hbm, dst_hbm, out_hbm,
                   src_v, dst_v, bufs, gsem, ssem, acc):
        c = lax.axis_index("c")
        s = lax.axis_index("s")
        row0 = c * n_nodes + s * rows_per_tile
        pltpu.sync_copy(
            y_hbm.at[pl.ds(row0, rows_per_tile)],
            acc.at[pl.ds(s * rows_per_tile, rows_per_tile)],
        )
        plsc.subcore_barrier()

        def g_fire(j, t):
            pltpu.async_copy(y_hbm.at[src_v.at[j]], bufs[t], gsem[t])

        def g_wait(j, t):
            pltpu.make_async_copy(y_hbm.at[src_v.at[j]], bufs[t],
                                  gsem[t]).wait()

        def sc_fire(j, t):
            pltpu.async_copy(bufs[t], acc.at[dst_v.at[j]], ssem[t], add=True)

        def sc_wait(j, t):
            pltpu.make_async_copy(bufs[t], acc.at[dst_v.at[j]],
                                  ssem[t]).wait()

        def body(k, _):
            j = 3 * k
            sc_wait(j - 1, 2); g_fire(j + 2, 2)
            g_wait(j, 0); sc_fire(j, 0)
            sc_wait(j, 0); g_fire(j + 3, 0)
            g_wait(j + 1, 1); sc_fire(j + 1, 1)
            sc_wait(j + 1, 1); g_fire(j + 4, 1)
            g_wait(j + 2, 2); sc_fire(j + 2, 2)
            return 0

        for w in range(2):
            pltpu.sync_copy(src_hbm.at[c, s, w], src_v)
            pltpu.sync_copy(dst_hbm.at[s, w], dst_v)
            g_fire(0, 0); g_fire(1, 1); g_fire(2, 2)
            g_wait(0, 0); sc_fire(0, 0); sc_wait(0, 0); g_fire(3, 0)
            g_wait(1, 1); sc_fire(1, 1); sc_wait(1, 1); g_fire(4, 1)
            g_wait(2, 2); sc_fire(2, 2)
            lax.fori_loop(1, W // 3 - 1, body, 0)
            e = W - 3
            sc_wait(e - 1, 2); g_fire(e + 2, 2)
            g_wait(e, 0); sc_fire(e, 0); sc_wait(e, 0)
            g_wait(e + 1, 1); sc_fire(e + 1, 1); sc_wait(e + 1, 1)
            g_wait(e + 2, 2); sc_fire(e + 2, 2); sc_wait(e + 2, 2)
        plsc.subcore_barrier()
        pltpu.sync_copy(
            acc.at[pl.ds(s * rows_per_tile, rows_per_tile)],
            out_hbm.at[pl.ds(row0, rows_per_tile)],
        )

    return agg_kernel


def _scale_kernel(n_nodes, n_pad, in_ch, half, acc_w, blk):
    def body(deg_ref, x_ref, dinv_ref, y_ref):
        degp1 = deg_ref[0][:, 0:1] + deg_ref[1][:, 0:1] - 1.0
        dinv = lax.rsqrt(degp1)
        dinv_ref[...] = jnp.broadcast_to(dinv, (blk, acc_w))
        y = x_ref[...] * dinv
        y_ref[0] = y[:, :half]
        y_ref[1] = y[:, half:]

    grid = n_nodes // blk
    return pl.pallas_call(
        body,
        grid=(grid,),
        in_specs=[
            pl.BlockSpec((2, blk, half), lambda i: (0, i, 0)),
            pl.BlockSpec((blk, in_ch), lambda i: (i, 0)),
        ],
        out_specs=[
            pl.BlockSpec((blk, acc_w), lambda i: (i, 0)),
            pl.BlockSpec((2, blk, half), lambda i: (0, i, 0)),
        ],
        out_shape=[
            jax.ShapeDtypeStruct((n_pad, acc_w), jnp.float32),
            jax.ShapeDtypeStruct((2, n_pad, half), jnp.float32),
        ],
    )


def _mlp_kernel(n_nodes, n_pad, in_ch, hid_ch, out_ch, half, acc_w, blk):
    def body(agg_ref, dinv_ref, w1_ref, b1_ref, w2_ref, z_ref):
        a = jnp.concatenate([agg_ref[0], agg_ref[1]], axis=1)
        dinv = dinv_ref[...][:, 0:1]
        a = a * dinv
        h = jnp.dot(a, w1_ref[...], preferred_element_type=jnp.float32,
                    precision=None)
        h = jnp.maximum(h + b1_ref[...], 0.0)
        z = jnp.dot(h, w2_ref[...], preferred_element_type=jnp.float32,
                    precision=None)
        z = z * dinv
        z_ref[0] = z[:, :half]
        z_ref[1] = z[:, half:]

    grid = n_nodes // blk
    return pl.pallas_call(
        body,
        grid=(grid,),
        in_specs=[
            pl.BlockSpec((2, blk, half), lambda i: (0, i, 0)),
            pl.BlockSpec((blk, acc_w), lambda i: (i, 0)),
            pl.BlockSpec((in_ch, hid_ch), lambda i: (0, 0)),
            pl.BlockSpec((1, hid_ch), lambda i: (0, 0)),
            pl.BlockSpec((hid_ch, out_ch), lambda i: (0, 0)),
        ],
        out_specs=pl.BlockSpec((2, blk, half), lambda i: (0, i, 0)),
        out_shape=jax.ShapeDtypeStruct((2, n_pad, half), jnp.float32),
    )


def _final_kernel(n_nodes, out_ch, half, acc_w, blk):
    def body(agg_ref, dinv_ref, b2_ref, out_ref):
        a = jnp.concatenate([agg_ref[0], agg_ref[1]], axis=1)
        dinv = dinv_ref[...][:, 0:1]
        out_ref[...] = a * dinv + b2_ref[...]

    grid = n_nodes // blk
    return pl.pallas_call(
        body,
        grid=(grid,),
        in_specs=[
            pl.BlockSpec((2, blk, half), lambda i: (0, i, 0)),
            pl.BlockSpec((blk, acc_w), lambda i: (i, 0)),
            pl.BlockSpec((1, out_ch), lambda i: (0, 0)),
        ],
        out_specs=pl.BlockSpec((blk, out_ch), lambda i: (i, 0)),
        out_shape=jax.ShapeDtypeStruct((n_nodes, out_ch), jnp.float32),
    )


def kernel(x, edge_index, W1, b1, W2, b2):
    n_nodes, in_ch = x.shape
    hid_ch = W1.shape[1]
    out_ch = W2.shape[1]
    n_edges = edge_index.shape[1]
    half = in_ch // 2
    acc_w = 8
    blk = 2000

    n_pad = -(-n_nodes // (NS * 8)) * (NS * 8)
    rows_per_tile = n_pad // NS
    per_tile = -(-n_edges // NS)
    n_chunks = -(-per_tile // CHUNK)
    n_chunks = (n_chunks + 5) // 6 * 6
    nacc = n_pad

    src = edge_index[0].astype(jnp.int32)
    dst = edge_index[1].astype(jnp.int32)
    e_pad = NS * n_chunks * CHUNK - n_edges
    src_p = jnp.concatenate([src, jnp.zeros((e_pad,), jnp.int32)])
    dst_p = jnp.concatenate([dst, jnp.full((e_pad,), n_nodes, jnp.int32)])
    W = n_chunks // 2
    src_t = src_p.reshape(NS, 2, W, CHUNK)
    src_both = jnp.stack([src_t, src_t + n_pad])
    dst_t = dst_p.reshape(NS, 2, W, CHUNK)

    agg = _make_agg_kernel(n_pad, n_chunks, nacc, rows_per_tile, half)
    ones = jnp.ones((rows_per_tile, half), jnp.float32)
    degp = _make_deg_kernel(n_pad, n_chunks, nacc, rows_per_tile, half)(
        ones, dst_t)
    dinv, ybuf = _scale_kernel(n_nodes, n_pad, in_ch, half, acc_w, blk)(
        degp.reshape(2, n_pad, half), x)

    agg1 = agg(ybuf.reshape(2 * n_pad, half), src_both, dst_t)
    z = _mlp_kernel(n_nodes, n_pad, in_ch, hid_ch, out_ch, half, acc_w, blk)(
        agg1.reshape(2, n_pad, half), dinv, W1, b1.reshape(1, hid_ch), W2)
    agg2 = agg(z.reshape(2 * n_pad, half), src_both, dst_t)
    return _final_kernel(n_nodes, out_ch, half, acc_w, blk)(
        agg2.reshape(2, n_pad, half), dinv, b2.reshape(1, out_ch))
